# baseline (speedup 1.0000x reference)
import functools

import jax
import jax.numpy as jnp
from jax import lax
from jax.experimental import pallas as pl
from jax.experimental.pallas import tpu as pltpu
from jax.experimental.pallas import tpu_sc as plsc

VOCAB = 1000
D = 2048
F = 8
B, S = 2, 2048
N = B * S

NC, NS, L = 2, 16, 16
NW = NC * NS
P_W = N // NW
C = 4
G = C * F
NCHUNK = P_W // C

_mesh = plsc.VectorSubcoreMesh(core_axis_name="c", subcore_axis_name="s")


@functools.partial(
    pl.kernel,
    mesh=_mesh,
    out_type=jax.ShapeDtypeStruct((N, D), jnp.float32),
    scratch_types=[
        pltpu.VMEM((NCHUNK, G), jnp.int32),
        pltpu.VMEM((G, D), jnp.float32),
        pltpu.VMEM((C, D), jnp.float32),
        pltpu.SemaphoreType.DMA,
    ],
)
def _embed_sc(x_hbm, w_hbm, out_hbm, idx_v, rows_v, acc_v, sem):
    wid = lax.axis_index("s") * NC + lax.axis_index("c")
    row_base = wid * P_W

    pltpu.sync_copy(x_hbm.at[wid], idx_v)
    offs = (lax.iota(jnp.int32, 16) % F) * VOCAB

    def _bias(c, carry):
        for h in (0, L):
            idx_v[c, pl.ds(h, L)] = idx_v[c, pl.ds(h, L)] + offs
        return carry

    lax.fori_loop(0, NCHUNK, _bias, 0)

    def _chunk(k, carry):
        pltpu.async_copy(w_hbm.at[idx_v.at[k]], rows_v, sem).wait()

        def _cols(j, inner):
            col = j * L
            for c in range(C):
                s = rows_v[c * F, pl.ds(col, L)]
                for f in range(1, F):
                    s = s + rows_v[c * F + f, pl.ds(col, L)]
                acc_v[c, pl.ds(col, L)] = s
            return inner

        lax.fori_loop(0, D // L, _cols, 0)
        pltpu.sync_copy(acc_v, out_hbm.at[pl.ds(row_base + k * C, C)])
        return carry

    lax.fori_loop(0, NCHUNK, _chunk, 0)


def kernel(x, W):
    xf = x.astype(jnp.int32).reshape(NW, NCHUNK, G)
    out = _embed_sc(xf, W.astype(jnp.float32))
    return out.reshape(B, S, D)


# speedup vs baseline: 1.3945x; 1.3945x over previous
import functools

import jax
import jax.numpy as jnp
from jax import lax
from jax.experimental import pallas as pl
from jax.experimental.pallas import tpu as pltpu
from jax.experimental.pallas import tpu_sc as plsc

VOCAB = 1000
D = 2048
F = 8
B, S = 2, 2048
N = B * S

NC, NS, L = 2, 16, 16
NW = NC * NS
P_W = N // NW
C = 2
G = C * F
NCHUNK = P_W // C

_mesh = plsc.VectorSubcoreMesh(core_axis_name="c", subcore_axis_name="s")


@functools.partial(
    pl.kernel,
    mesh=_mesh,
    out_type=jax.ShapeDtypeStruct((N, D), jnp.float32),
    scratch_types=[
        pltpu.VMEM((NCHUNK, G), jnp.int32),
        pltpu.VMEM((G, D), jnp.float32),
        pltpu.VMEM((G, D), jnp.float32),
        pltpu.VMEM((C, D), jnp.float32),
        pltpu.SemaphoreType.DMA,
        pltpu.SemaphoreType.DMA,
    ],
)
def _embed_sc(x_hbm, w_hbm, out_hbm, idx_v, rows0, rows1, acc_v, sem0, sem1):
    wid = lax.axis_index("s") * NC + lax.axis_index("c")
    row_base = wid * P_W
    bufs = (rows0, rows1)
    sems = (sem0, sem1)

    pltpu.sync_copy(x_hbm.at[wid], idx_v)
    offs = (lax.iota(jnp.int32, 16) % F) * VOCAB

    def _bias(c, carry):
        idx_v[c, pl.ds(0, L)] = idx_v[c, pl.ds(0, L)] + offs
        return carry

    lax.fori_loop(0, NCHUNK, _bias, 0)

    pltpu.async_copy(w_hbm.at[idx_v.at[0]], rows0, sem0)

    def _pair(k2, carry):
        for b in range(2):
            k = k2 * 2 + b
            nb = 1 - b

            @pl.when(k + 1 < NCHUNK)
            def _():
                pltpu.async_copy(w_hbm.at[idx_v.at[k + 1]], bufs[nb], sems[nb])

            pltpu.make_async_copy(w_hbm.at[idx_v.at[k]], bufs[b], sems[b]).wait()
            rows_v = bufs[b]

            def _cols(j, inner):
                col = j * L
                for c in range(C):
                    s = rows_v[c * F, pl.ds(col, L)]
                    for f in range(1, F):
                        s = s + rows_v[c * F + f, pl.ds(col, L)]
                    acc_v[c, pl.ds(col, L)] = s
                return inner

            lax.fori_loop(0, D // L, _cols, 0)
            pltpu.sync_copy(acc_v, out_hbm.at[pl.ds(row_base + k * C, C)])
        return carry

    lax.fori_loop(0, NCHUNK // 2, _pair, 0)


def kernel(x, W):
    xf = x.astype(jnp.int32).reshape(NW, NCHUNK, G)
    out = _embed_sc(xf, W.astype(jnp.float32))
    return out.reshape(B, S, D)


# speedup vs baseline: 2.4410x; 1.7504x over previous
import functools

import jax
import jax.numpy as jnp
from jax import lax
from jax.experimental import pallas as pl
from jax.experimental.pallas import tpu as pltpu
from jax.experimental.pallas import tpu_sc as plsc

VOCAB = 1000
D = 2048
F = 8
B, S = 2, 2048
N = B * S

NC, NS, L = 2, 16, 16
NW = NC * NS
P_W = N // NW
C = 2
G = C * F
NCHUNK = P_W // C

_mesh = plsc.VectorSubcoreMesh(core_axis_name="c", subcore_axis_name="s")


@functools.partial(
    pl.kernel,
    mesh=_mesh,
    out_type=jax.ShapeDtypeStruct((N, D), jnp.float32),
    scratch_types=[
        pltpu.VMEM((NCHUNK, G), jnp.int32),
        pltpu.VMEM((G, D), jnp.float32),
        pltpu.VMEM((G, D), jnp.float32),
        pltpu.VMEM((C, D), jnp.float32),
        pltpu.SemaphoreType.DMA,
        pltpu.SemaphoreType.DMA,
    ],
)
def _embed_sc(x_hbm, w_hbm, out_hbm, idx_v, rows0, rows1, acc_v, sem0, sem1):
    wid = lax.axis_index("s") * NC + lax.axis_index("c")
    row_base = wid * P_W
    bufs = (rows0, rows1)
    sems = (sem0, sem1)

    pltpu.sync_copy(x_hbm.at[wid], idx_v)
    offs = (lax.iota(jnp.int32, 16) % F) * VOCAB

    def _bias(c, carry):
        idx_v[c, pl.ds(0, L)] = idx_v[c, pl.ds(0, L)] + offs
        return carry

    lax.fori_loop(0, NCHUNK, _bias, 0)

    pltpu.async_copy(w_hbm.at[idx_v.at[0]], rows0, sem0)

    def _pair(k2, carry):
        for b in range(2):
            k = k2 * 2 + b
            nb = 1 - b

            @pl.when(k + 1 < NCHUNK)
            def _():
                pltpu.async_copy(w_hbm.at[idx_v.at[k + 1]], bufs[nb], sems[nb])

            pltpu.make_async_copy(w_hbm.at[idx_v.at[k]], bufs[b], sems[b]).wait()
            rows_v = bufs[b]

            @plsc.parallel_loop(0, D // L, unroll=4)
            def _cols(j):
                col = j * L
                for c in range(C):
                    t = [rows_v[c * F + f, pl.ds(col, L)] for f in range(F)]
                    while len(t) > 1:
                        t = [a + b2 for a, b2 in zip(t[::2], t[1::2])]
                    acc_v[c, pl.ds(col, L)] = t[0]
            pltpu.sync_copy(acc_v, out_hbm.at[pl.ds(row_base + k * C, C)])
        return carry

    lax.fori_loop(0, NCHUNK // 2, _pair, 0)


def kernel(x, W):
    xf = x.astype(jnp.int32).reshape(NW, NCHUNK, G)
    out = _embed_sc(xf, W.astype(jnp.float32))
    return out.reshape(B, S, D)


# speedup vs baseline: 2.4438x; 1.0011x over previous
import functools

import jax
import jax.numpy as jnp
from jax import lax
from jax.experimental import pallas as pl
from jax.experimental.pallas import tpu as pltpu
from jax.experimental.pallas import tpu_sc as plsc

VOCAB = 1000
D = 2048
F = 8
B, S = 2, 2048
N = B * S

NC, NS, L = 2, 16, 16
NW = NC * NS
P_W = N // NW
C = 2
G = C * F
NCHUNK = P_W // C

_mesh = plsc.VectorSubcoreMesh(core_axis_name="c", subcore_axis_name="s")


@functools.partial(
    pl.kernel,
    mesh=_mesh,
    out_type=jax.ShapeDtypeStruct((N, D), jnp.float32),
    scratch_types=[
        pltpu.VMEM((NCHUNK, G), jnp.int32),
        pltpu.VMEM((G, D), jnp.float32),
        pltpu.VMEM((G, D), jnp.float32),
        pltpu.VMEM((C, D), jnp.float32),
        pltpu.SemaphoreType.DMA,
        pltpu.SemaphoreType.DMA,
    ],
)
def _embed_sc(x_hbm, w_hbm, out_hbm, idx_v, rows0, rows1, acc_v, sem0, sem1):
    wid = lax.axis_index("s") * NC + lax.axis_index("c")
    row_base = wid * P_W
    bufs = (rows0, rows1)
    sems = (sem0, sem1)

    pltpu.sync_copy(x_hbm.at[wid], idx_v)
    offs = (lax.iota(jnp.int32, 16) % F) * VOCAB

    def _bias(c, carry):
        idx_v[c, pl.ds(0, L)] = idx_v[c, pl.ds(0, L)] + offs
        return carry

    lax.fori_loop(0, NCHUNK, _bias, 0)

    pltpu.async_copy(w_hbm.at[idx_v.at[0]], rows0, sem0)

    def _pair(k2, carry):
        for b in range(2):
            k = k2 * 2 + b
            nb = 1 - b

            @pl.when(k + 1 < NCHUNK)
            def _():
                pltpu.async_copy(w_hbm.at[idx_v.at[k + 1]], bufs[nb], sems[nb])

            pltpu.make_async_copy(w_hbm.at[idx_v.at[k]], bufs[b], sems[b]).wait()
            rows_v = bufs[b]

            @plsc.parallel_loop(0, D // L, unroll=4)
            def _cols(j):
                col = j * L
                for c in range(C):
                    t = [rows_v[c * F + f, pl.ds(col, L)] for f in range(F)]
                    while len(t) > 1:
                        t = [a + b2 for a, b2 in zip(t[::2], t[1::2])]
                    acc_v[c, pl.ds(col, L)] = t[0]

            pltpu.sync_copy(acc_v, out_hbm.at[pl.ds(row_base + k * C, C)])
        return carry

    lax.fori_loop(0, NCHUNK // 2, _pair, 0)


def kernel(x, W):
    xf = x.astype(jnp.int32).reshape(NW, NCHUNK, G)
    out = _embed_sc(xf, W)
    return out.reshape(B, S, D)


# speedup vs baseline: 2.5301x; 1.0353x over previous
import functools

import jax
import jax.numpy as jnp
from jax import lax
from jax.experimental import pallas as pl
from jax.experimental.pallas import tpu as pltpu
from jax.experimental.pallas import tpu_sc as plsc

VOCAB = 1000
D = 2048
F = 8
B, S = 2, 2048
N = B * S

NC, NS, L = 2, 16, 16
NW = NC * NS
P_W = N // NW
C = 2
G = C * F
NCHUNK = P_W // C

_mesh = plsc.VectorSubcoreMesh(core_axis_name="c", subcore_axis_name="s")


@functools.partial(
    pl.kernel,
    mesh=_mesh,
    out_type=jax.ShapeDtypeStruct((N, D), jnp.float32),
    scratch_types=[
        pltpu.VMEM((NCHUNK, G), jnp.int32),
        pltpu.VMEM((G, D), jnp.float32),
        pltpu.VMEM((G, D), jnp.float32),
        pltpu.VMEM((C, D), jnp.float32),
        pltpu.VMEM((C, D), jnp.float32),
        pltpu.SemaphoreType.DMA,
        pltpu.SemaphoreType.DMA,
        pltpu.SemaphoreType.DMA,
        pltpu.SemaphoreType.DMA,
    ],
)
def _embed_sc(x_hbm, w_hbm, out_hbm, idx_v, rows0, rows1,
              acc0, acc1, sem0, sem1, wsem0, wsem1):
    wid = lax.axis_index("s") * NC + lax.axis_index("c")
    row_base = wid * P_W
    bufs = (rows0, rows1)
    sems = (sem0, sem1)
    accs = (acc0, acc1)
    wsems = (wsem0, wsem1)

    pltpu.sync_copy(x_hbm.at[wid], idx_v)
    offs = (lax.iota(jnp.int32, 16) % F) * VOCAB

    def _bias(c, carry):
        idx_v[c, pl.ds(0, L)] = idx_v[c, pl.ds(0, L)] + offs
        return carry

    lax.fori_loop(0, NCHUNK, _bias, 0)

    pltpu.async_copy(w_hbm.at[idx_v.at[0]], rows0, sem0)

    def _pair(k2, carry):
        for b in range(2):
            k = k2 * 2 + b
            nb = 1 - b

            @pl.when(k + 1 < NCHUNK)
            def _():
                pltpu.async_copy(w_hbm.at[idx_v.at[k + 1]], bufs[nb], sems[nb])

            pltpu.make_async_copy(w_hbm.at[idx_v.at[k]], bufs[b], sems[b]).wait()
            rows_v = bufs[b]
            acc_v = accs[b]

            @pl.when(k >= 2)
            def _():
                pltpu.make_async_copy(
                    acc_v, out_hbm.at[pl.ds(row_base + (k - 2) * C, C)], wsems[b]
                ).wait()

            @plsc.parallel_loop(0, D // L, unroll=8)
            def _cols(j):
                col = j * L
                for c in range(C):
                    t = [rows_v[c * F + f, pl.ds(col, L)] for f in range(F)]
                    while len(t) > 1:
                        t = [a + b2 for a, b2 in zip(t[::2], t[1::2])]
                    acc_v[c, pl.ds(col, L)] = t[0]

            pltpu.async_copy(
                acc_v, out_hbm.at[pl.ds(row_base + k * C, C)], wsems[b]
            )
        return carry

    lax.fori_loop(0, NCHUNK // 2, _pair, 0)

    for b in range(2):
        k = NCHUNK - 2 + b
        pltpu.make_async_copy(
            accs[b], out_hbm.at[pl.ds(row_base + k * C, C)], wsems[b]
        ).wait()


def kernel(x, W):
    xf = x.astype(jnp.int32).reshape(NW, NCHUNK, G)
    out = _embed_sc(xf, W)
    return out.reshape(B, S, D)
